# baseline (speedup 1.0000x reference)
import jax
import jax.numpy as jnp
from jax.experimental import pallas as pl
from jax.experimental.pallas import tpu as pltpu
from jax.experimental.pallas import tpu_sc as plsc

_KP_EXTENT = 1.2
_GATHER_WINDOW = 256
_CHUNKS = 2
_BLOCK = 200


def _sc_gather(table, inds_flat, NH, W):
    mesh = plsc.VectorSubcoreMesh(core_axis_name="c", subcore_axis_name="s")
    G = _GATHER_WINDOW

    @pl.kernel(
        out_type=jax.ShapeDtypeStruct((NH, W), table.dtype),
        mesh=mesh,
    )
    def gather_kernel(t_hbm, i_hbm, o_hbm):
        def body(i_vmem, o_vmem):
            pltpu.sync_copy(t_hbm.at[i_vmem.at[0]], o_vmem)

        pltpu.emit_pipeline(
            body,
            grid=(NH // G,),
            in_specs=[pl.BlockSpec((1, G), lambda i: (0, i))],
            out_specs=[pl.BlockSpec((G, W), lambda i: (i, 0))],
            core_axis_name=("c", "s"),
            dimension_semantics=(pltpu.PARALLEL,),
        )(i_hbm, o_hbm)

    return gather_kernel(table, inds_flat)


def _mask_hi(xf32):
    bits = jax.lax.bitcast_convert_type(xf32, jnp.int32)
    return jax.lax.bitcast_convert_type(
        jnp.bitwise_and(bits, jnp.int32(-65536)), jnp.float32)


def _tc_compute(xgc, q3, Vh, Vl, dwA, dwB, himask, bias2, Nc, H, C, B):
    Ch = C // 2

    def body(xgc_ref, q_ref, vh_ref, vl_ref, dwa_ref, dwb_ref,
             m_ref, b_ref, o_ref):
        sg = xgc_ref[:, :8].reshape(B, H, 8)
        rel = sg - q_ref[...]
        nrm2 = jnp.sum(rel * rel, axis=2)
        relm = rel.reshape(B * H, 8)
        hi_f = _mask_hi(relm)
        rh = hi_f.astype(jnp.bfloat16)
        rl = (relm - hi_f).astype(jnp.bfloat16)
        f32 = jnp.float32
        dots = (jnp.dot(rh, vh_ref[...], preferred_element_type=f32)
                + jnp.dot(rh, vl_ref[...], preferred_element_type=f32)
                + jnp.dot(rl, vh_ref[...], preferred_element_type=f32))
        d2 = jnp.maximum(dots + nrm2.reshape(B * H, 1), 0.0)
        w = jnp.maximum(1.0 - jnp.sqrt(d2) * (1.0 / _KP_EXTENT), 0.0)
        wb = w.astype(jnp.bfloat16)
        a_hi = jnp.dot(wb, dwa_ref[...], preferred_element_type=f32)
        a_lo = jnp.dot(wb, dwb_ref[...], preferred_element_type=f32)
        fbits = jax.lax.bitcast_convert_type(xgc_ref[...], jnp.int32)
        f_hi = jax.lax.bitcast_convert_type(
            jnp.bitwise_and(fbits, m_ref[...]), f32)
        f_lo = jax.lax.bitcast_convert_type(
            jnp.left_shift(fbits, 16), f32)
        r1 = jnp.sum((a_hi * f_hi).reshape(B, H, C), axis=1)
        r2 = jnp.sum((a_lo * f_lo).reshape(B, H, C), axis=1)
        o_ref[:, :Ch] = r1[:, Ch:] + b_ref[:, :Ch]
        o_ref[:, Ch:] = r2[:, Ch:] + b_ref[:, Ch:]

    return pl.pallas_call(
        body,
        grid=(Nc // B,),
        in_specs=[
            pl.BlockSpec((B * H, C), lambda i: (i, 0)),
            pl.BlockSpec((B, 1, 8), lambda i: (i, 0, 0)),
            pl.BlockSpec((8, 16), lambda i: (0, 0)),
            pl.BlockSpec((8, 16), lambda i: (0, 0)),
            pl.BlockSpec((16, C), lambda i: (0, 0)),
            pl.BlockSpec((16, C), lambda i: (0, 0)),
            pl.BlockSpec((1, C), lambda i: (0, 0)),
            pl.BlockSpec((1, C), lambda i: (0, 0)),
        ],
        out_specs=pl.BlockSpec((B, C), lambda i: (i, 0)),
        out_shape=jax.ShapeDtypeStruct((Nc, C), jnp.float32),
    )(xgc, q3, Vh, Vl, dwA, dwB, himask, bias2)


def kernel(q_pts, s_pts, neighb_inds, x, kernel_points, dw_weights, bias):
    N, H = neighb_inds.shape
    M, C = x.shape
    K = kernel_points.shape[0]
    Ch = C // 2

    x_p = jnp.concatenate([x, jnp.zeros((1, C), x.dtype)], axis=0)
    hi16 = jax.lax.bitcast_convert_type(
        x_p[:, :Ch].astype(jnp.bfloat16), jnp.uint16).astype(jnp.uint32)
    lo16 = jax.lax.bitcast_convert_type(
        x_p[:, Ch:].astype(jnp.bfloat16), jnp.uint16).astype(jnp.uint32)
    packed = jax.lax.bitcast_convert_type(
        jnp.left_shift(hi16, 16) | lo16, jnp.float32)
    s_pad = jnp.concatenate([s_pts, jnp.full((1, 3), 1e6, s_pts.dtype)], axis=0)
    table = jnp.concatenate(
        [s_pad, jnp.zeros((M + 1, Ch - 3), jnp.float32), packed], axis=1)

    q3 = jnp.concatenate(
        [q_pts, jnp.full((N, 1), -1.0, jnp.float32),
         jnp.zeros((N, 4), jnp.float32)], axis=1).reshape(N, 1, 8)

    crow = jnp.concatenate(
        [jnp.sum(kernel_points * kernel_points, axis=1) - 1.0,
         jnp.full((16 - K,), 1e12, jnp.float32)]).reshape(1, 16)
    V = jnp.zeros((8, 16), jnp.float32).at[:3, :K].set(-2.0 * kernel_points.T)
    V = V.at[3:4, :].set(crow)
    V_hi = _mask_hi(V)
    Vh = V_hi.astype(jnp.bfloat16)
    Vl = (V - V_hi).astype(jnp.bfloat16)
    dwp = jnp.pad(dw_weights, ((0, 16 - K), (0, 0)))
    zeroL = jnp.zeros((16, Ch), jnp.float32)
    dwA = jnp.concatenate([zeroL, dwp[:, :Ch]], axis=1).astype(jnp.bfloat16)
    dwB = jnp.concatenate([zeroL, dwp[:, Ch:]], axis=1).astype(jnp.bfloat16)
    himask = jnp.concatenate(
        [jnp.zeros((1, Ch), jnp.int32),
         jnp.full((1, Ch), -65536, jnp.int32)], axis=1)
    bias2 = bias.reshape(1, C)

    Nc = N // _CHUNKS
    gathered = []
    for c in range(_CHUNKS):
        inds_c = neighb_inds[c * Nc:(c + 1) * Nc].reshape(1, Nc * H)
        gathered.append(_sc_gather(table, inds_c, Nc * H, C))
    outs = []
    for c in range(_CHUNKS):
        outs.append(_tc_compute(gathered[c], q3[c * Nc:(c + 1) * Nc], Vh, Vl,
                                dwA, dwB, himask, bias2, Nc, H, C, _BLOCK))
    return jnp.concatenate(outs, axis=0)


# speedup vs baseline: 1.1273x; 1.1273x over previous
import jax
import jax.numpy as jnp
from jax.experimental import pallas as pl
from jax.experimental.pallas import tpu as pltpu
from jax.experimental.pallas import tpu_sc as plsc

_KP_EXTENT = 1.2
_GATHER_WINDOW = 256
_CHUNKS = 5
_BLOCK = 400


def _sc_gather(table, inds_flat, NH, W):
    mesh = plsc.VectorSubcoreMesh(core_axis_name="c", subcore_axis_name="s")
    G = _GATHER_WINDOW

    @pl.kernel(
        out_type=jax.ShapeDtypeStruct((NH, W), table.dtype),
        mesh=mesh,
    )
    def gather_kernel(t_hbm, i_hbm, o_hbm):
        def body(i_vmem, o_vmem):
            pltpu.sync_copy(t_hbm.at[i_vmem.at[0]], o_vmem)

        pltpu.emit_pipeline(
            body,
            grid=(NH // G,),
            in_specs=[pl.BlockSpec((1, G), lambda i: (0, i))],
            out_specs=[pl.BlockSpec((G, W), lambda i: (i, 0))],
            core_axis_name=("c", "s"),
            dimension_semantics=(pltpu.PARALLEL,),
        )(i_hbm, o_hbm)

    return gather_kernel(table, inds_flat)


def _mask_hi(xf32):
    bits = jax.lax.bitcast_convert_type(xf32, jnp.int32)
    return jax.lax.bitcast_convert_type(
        jnp.bitwise_and(bits, jnp.int32(-65536)), jnp.float32)


def _tc_compute(xgc, q3, Vh, Vl, dwA, dwB, himask, bias2, Nc, H, C, B):
    Ch = C // 2

    def body(xgc_ref, q_ref, vh_ref, vl_ref, dwa_ref, dwb_ref,
             m_ref, b_ref, o_ref):
        sg = xgc_ref[:, :8].reshape(B, H, 8)
        rel = sg - q_ref[...]
        nrm2 = jnp.sum(rel * rel, axis=2)
        relm = rel.reshape(B * H, 8)
        hi_f = _mask_hi(relm)
        rh = hi_f.astype(jnp.bfloat16)
        rl = (relm - hi_f).astype(jnp.bfloat16)
        f32 = jnp.float32
        dots = (jnp.dot(rh, vh_ref[...], preferred_element_type=f32)
                + jnp.dot(rh, vl_ref[...], preferred_element_type=f32)
                + jnp.dot(rl, vh_ref[...], preferred_element_type=f32))
        d2 = jnp.maximum(dots + nrm2.reshape(B * H, 1), 0.0)
        w = jnp.maximum(1.0 - jnp.sqrt(d2) * (1.0 / _KP_EXTENT), 0.0)
        wb = w.astype(jnp.bfloat16)
        a_hi = jnp.dot(wb, dwa_ref[...], preferred_element_type=f32)
        a_lo = jnp.dot(wb, dwb_ref[...], preferred_element_type=f32)
        fbits = jax.lax.bitcast_convert_type(xgc_ref[...], jnp.int32)
        f_hi = jax.lax.bitcast_convert_type(
            jnp.bitwise_and(fbits, m_ref[...]), f32)
        f_lo = jax.lax.bitcast_convert_type(
            jnp.left_shift(fbits, 16), f32)
        r1 = jnp.sum((a_hi * f_hi).reshape(B, H, C), axis=1)
        r2 = jnp.sum((a_lo * f_lo).reshape(B, H, C), axis=1)
        o_ref[:, :Ch] = r1[:, Ch:] + b_ref[:, :Ch]
        o_ref[:, Ch:] = r2[:, Ch:] + b_ref[:, Ch:]

    return pl.pallas_call(
        body,
        grid=(Nc // B,),
        in_specs=[
            pl.BlockSpec((B * H, C), lambda i: (i, 0)),
            pl.BlockSpec((B, 1, 8), lambda i: (i, 0, 0)),
            pl.BlockSpec((8, 16), lambda i: (0, 0)),
            pl.BlockSpec((8, 16), lambda i: (0, 0)),
            pl.BlockSpec((16, C), lambda i: (0, 0)),
            pl.BlockSpec((16, C), lambda i: (0, 0)),
            pl.BlockSpec((1, C), lambda i: (0, 0)),
            pl.BlockSpec((1, C), lambda i: (0, 0)),
        ],
        out_specs=pl.BlockSpec((B, C), lambda i: (i, 0)),
        out_shape=jax.ShapeDtypeStruct((Nc, C), jnp.float32),
    )(xgc, q3, Vh, Vl, dwA, dwB, himask, bias2)


def kernel(q_pts, s_pts, neighb_inds, x, kernel_points, dw_weights, bias):
    N, H = neighb_inds.shape
    M, C = x.shape
    K = kernel_points.shape[0]
    Ch = C // 2

    x_p = jnp.concatenate([x, jnp.zeros((1, C), x.dtype)], axis=0)
    hi16 = jax.lax.bitcast_convert_type(
        x_p[:, :Ch].astype(jnp.bfloat16), jnp.uint16).astype(jnp.uint32)
    lo16 = jax.lax.bitcast_convert_type(
        x_p[:, Ch:].astype(jnp.bfloat16), jnp.uint16).astype(jnp.uint32)
    packed = jax.lax.bitcast_convert_type(
        jnp.left_shift(hi16, 16) | lo16, jnp.float32)
    s_pad = jnp.concatenate([s_pts, jnp.full((1, 3), 1e6, s_pts.dtype)], axis=0)
    table = jnp.concatenate(
        [s_pad, jnp.zeros((M + 1, Ch - 3), jnp.float32), packed], axis=1)

    q3 = jnp.concatenate(
        [q_pts, jnp.full((N, 1), -1.0, jnp.float32),
         jnp.zeros((N, 4), jnp.float32)], axis=1).reshape(N, 1, 8)

    crow = jnp.concatenate(
        [jnp.sum(kernel_points * kernel_points, axis=1) - 1.0,
         jnp.full((16 - K,), 1e12, jnp.float32)]).reshape(1, 16)
    V = jnp.zeros((8, 16), jnp.float32).at[:3, :K].set(-2.0 * kernel_points.T)
    V = V.at[3:4, :].set(crow)
    V_hi = _mask_hi(V)
    Vh = V_hi.astype(jnp.bfloat16)
    Vl = (V - V_hi).astype(jnp.bfloat16)
    dwp = jnp.pad(dw_weights, ((0, 16 - K), (0, 0)))
    zeroL = jnp.zeros((16, Ch), jnp.float32)
    dwA = jnp.concatenate([zeroL, dwp[:, :Ch]], axis=1).astype(jnp.bfloat16)
    dwB = jnp.concatenate([zeroL, dwp[:, Ch:]], axis=1).astype(jnp.bfloat16)
    himask = jnp.concatenate(
        [jnp.zeros((1, Ch), jnp.int32),
         jnp.full((1, Ch), -65536, jnp.int32)], axis=1)
    bias2 = bias.reshape(1, C)

    bounds = [0, 1200, 3400, 5600, 7800, 10000]
    gathered = []
    for lo, hi in zip(bounds[:-1], bounds[1:]):
        inds_c = neighb_inds[lo:hi].reshape(1, (hi - lo) * H)
        gathered.append(_sc_gather(table, inds_c, (hi - lo) * H, C))
    outs = []
    for (lo, hi), g in zip(zip(bounds[:-1], bounds[1:]), gathered):
        outs.append(_tc_compute(g, q3[lo:hi], Vh, Vl,
                                dwA, dwB, himask, bias2, hi - lo, H, C, _BLOCK))
    return jnp.concatenate(outs, axis=0)
